# baseline (speedup 1.0000x reference)
import functools

import jax
import jax.numpy as jnp
from jax import lax
from jax.experimental import pallas as pl
from jax.experimental.pallas import tpu as pltpu
from jax.experimental.pallas import tpu_sc as plsc

_R = 128
_C = 32768
_L = 16
_NW = 32

_RPW = 2
_K_SC = _NW * _RPW
_G = 16 // _RPW
_GSHIFT = {1: 0, 2: 1, 4: 2}[_RPW]

_NCHUNK = _C // _L
_NACC = 8
_NITER = _NCHUNK // _NACC


def _row_argmax(buf):
    lanes = lax.iota(jnp.int32, _L)

    def step(t, carry):
        ms, cs = carry
        tvec = lax.broadcast(t, (_L,))
        base = t * (_L * _NACC)
        nms, ncs = [], []
        for k in range(_NACC):
            x = buf[pl.ds(base + k * _L, _L)]
            gt = x > ms[k]
            nms.append(jnp.where(gt, x, ms[k]))
            ncs.append(jnp.where(gt, tvec, cs[k]))
        return tuple(nms), tuple(ncs)

    ms0 = tuple(buf[pl.ds(k * _L, _L)] for k in range(_NACC))
    cs0 = tuple(jnp.zeros((_L,), jnp.int32) for _ in range(_NACC))
    ms, cs = lax.fori_loop(1, _NITER, step, (ms0, cs0), unroll=4)

    bm = ms[0]
    bidx = cs[0] * (_NACC * _L) + lanes
    for k in range(1, _NACC):
        idxk = cs[k] * (_NACC * _L) + (k * _L) + lanes
        better = (ms[k] > bm) | ((ms[k] == bm) & (idxk < bidx))
        bm = jnp.where(better, ms[k], bm)
        bidx = jnp.where(better, idxk, bidx)
    sm = bm[0]
    si = bidx[0]
    for j in range(1, _L):
        mv = bm[j]
        iv = bidx[j]
        better = (mv > sm) | ((mv == sm) & (iv < si))
        sm = jnp.where(better, mv, sm)
        si = jnp.where(better, iv, si)
    return si


@functools.partial(
    pl.kernel,
    mesh=plsc.VectorSubcoreMesh(core_axis_name="c", subcore_axis_name="s"),
    out_type=jax.ShapeDtypeStruct((_K_SC,), jnp.int32),
    scratch_types=[
        pltpu.VMEM((_C,), jnp.float32),
        pltpu.VMEM((_C,), jnp.float32),
        pltpu.VMEM((_L,), jnp.int32),
        pltpu.VMEM((_G * _L,), jnp.int32),
        pltpu.VMEM((_L,), jnp.int32),
        pltpu.VMEM_SHARED((16 * _L,), jnp.int32),
        pltpu.SemaphoreType.DMA,
        pltpu.SemaphoreType.DMA,
    ],
)
def _argmax_sc(x_hbm, out_hbm, buf0, buf1, res, gbuf, outv, shared,
               sem0, sem1):
    cid = lax.axis_index("c")
    sid = lax.axis_index("s")
    wid = cid * 16 + sid
    row0 = wid * _RPW
    bufs = (buf0, buf1)
    sems = (sem0, sem1)
    lanes = lax.iota(jnp.int32, _L)

    copies = [None, None]
    copies[0] = pltpu.async_copy(x_hbm.at[row0], buf0, sem0)
    resvec = jnp.zeros((_L,), jnp.int32)
    for r in range(_RPW):
        cur = r % 2
        nxt = (r + 1) % 2
        if r + 1 < _RPW:
            copies[nxt] = pltpu.async_copy(
                x_hbm.at[row0 + r + 1], bufs[nxt], sems[nxt])
        copies[cur].wait()
        a = _row_argmax(bufs[cur])
        resvec = jnp.where(lanes == (sid % _G) * _RPW + r, a, resvec)
    res[...] = resvec
    pltpu.sync_copy(res, shared.at[pl.ds(pl.multiple_of(sid * _L, _L), _L)])
    plsc.subcore_barrier()

    grp = (sid // _G) * _G
    pltpu.sync_copy(
        shared.at[pl.ds(pl.multiple_of(grp * _L, _G * _L), _G * _L)], gbuf)
    hi = lanes >> _GSHIFT
    combined = gbuf[pl.ds(0, _L)]
    for j in range(1, _G):
        rowj = gbuf[pl.ds(j * _L, _L)]
        combined = jnp.where(hi == j, rowj, combined)
    outv[...] = combined

    @pl.when(sid == grp)
    def _():
        off = pl.multiple_of((cid * 16 + sid) * _RPW, _L)
        pltpu.sync_copy(outv, out_hbm.at[pl.ds(off, _L)])


_TCR = 8
_TCB = 128
_TCACC = 4


def _tc_body(x_ref, o_ref):
    niter = x_ref.shape[1] // (_TCB * _TCACC)
    lane = lax.broadcasted_iota(jnp.int32, (_TCR, _TCB), 1)

    ms = [x_ref[:, pl.ds(k * _TCB, _TCB)] for k in range(_TCACC)]
    cs = [jnp.zeros((_TCR, _TCB), jnp.int32) for _ in range(_TCACC)]
    for t in range(1, niter):
        base = t * (_TCB * _TCACC)
        for k in range(_TCACC):
            x = x_ref[:, pl.ds(base + k * _TCB, _TCB)]
            gt = x > ms[k]
            ms[k] = jnp.where(gt, x, ms[k])
            cs[k] = jnp.where(gt, t, cs[k])

    stride = _TCACC * _TCB
    bm = ms[0]
    bidx = cs[0] * stride + lane
    for k in range(1, _TCACC):
        idxk = cs[k] * stride + (k * _TCB) + lane
        better = (ms[k] > bm) | ((ms[k] == bm) & (idxk < bidx))
        bm = jnp.where(better, ms[k], bm)
        bidx = jnp.where(better, idxk, bidx)
    rm = jnp.max(bm, axis=1)
    cand = jnp.where(bm == rm[:, None], bidx, jnp.int32(2 ** 30))
    o_ref[0, 0, :] = jnp.min(cand, axis=1)


def _argmax_tc(x, row_start, rows):
    grid = rows // _TCR
    blk0 = row_start // _TCR
    out = pl.pallas_call(
        _tc_body,
        grid=(grid,),
        in_specs=[pl.BlockSpec((_TCR, _C), lambda i: (i + blk0, 0))],
        out_specs=pl.BlockSpec((1, 1, _TCR), lambda i: (i, 0, 0)),
        out_shape=jax.ShapeDtypeStruct((grid, 1, _TCR), jnp.int32),
    )(x)
    return out.reshape(rows)


def kernel(tensor):
    sc_out = _argmax_sc(tensor)
    tc_out = _argmax_tc(tensor, _K_SC, _R - _K_SC)
    return jnp.concatenate(
        [sc_out.reshape(1, _K_SC), tc_out.reshape(1, _R - _K_SC)],
        axis=1).reshape(_R)


# speedup vs baseline: 1.0088x; 1.0088x over previous
import functools

import jax
import jax.numpy as jnp
from jax import lax
from jax.experimental import pallas as pl
from jax.experimental.pallas import tpu as pltpu
from jax.experimental.pallas import tpu_sc as plsc

_R = 128
_C = 32768
_L = 16
_NW = 32

_RPW = 2
_K_SC = _NW * _RPW
_G = 16 // _RPW
_GSHIFT = {1: 0, 2: 1, 4: 2}[_RPW]

_NCHUNK = _C // _L
_NACC = 4
_NITER = _NCHUNK // _NACC


def _row_argmax(buf):
    lanes = lax.iota(jnp.int32, _L)

    def step(t, carry):
        ms, cs = carry
        tvec = lax.broadcast(t, (_L,))
        base = t * (_L * _NACC)
        nms, ncs = [], []
        for k in range(_NACC):
            x = buf[pl.ds(base + k * _L, _L)]
            gt = x > ms[k]
            nms.append(jnp.where(gt, x, ms[k]))
            ncs.append(jnp.where(gt, tvec, cs[k]))
        return tuple(nms), tuple(ncs)

    ms0 = tuple(buf[pl.ds(k * _L, _L)] for k in range(_NACC))
    cs0 = tuple(jnp.zeros((_L,), jnp.int32) for _ in range(_NACC))
    ms, cs = lax.fori_loop(1, _NITER, step, (ms0, cs0), unroll=8)

    bm = ms[0]
    bidx = cs[0] * (_NACC * _L) + lanes
    for k in range(1, _NACC):
        idxk = cs[k] * (_NACC * _L) + (k * _L) + lanes
        better = (ms[k] > bm) | ((ms[k] == bm) & (idxk < bidx))
        bm = jnp.where(better, ms[k], bm)
        bidx = jnp.where(better, idxk, bidx)
    sm = bm[0]
    si = bidx[0]
    for j in range(1, _L):
        mv = bm[j]
        iv = bidx[j]
        better = (mv > sm) | ((mv == sm) & (iv < si))
        sm = jnp.where(better, mv, sm)
        si = jnp.where(better, iv, si)
    return si


@functools.partial(
    pl.kernel,
    mesh=plsc.VectorSubcoreMesh(core_axis_name="c", subcore_axis_name="s"),
    out_type=jax.ShapeDtypeStruct((_K_SC,), jnp.int32),
    scratch_types=[
        pltpu.VMEM((_C,), jnp.float32),
        pltpu.VMEM((_C,), jnp.float32),
        pltpu.VMEM((_L,), jnp.int32),
        pltpu.VMEM((_G * _L,), jnp.int32),
        pltpu.VMEM((_L,), jnp.int32),
        pltpu.VMEM_SHARED((16 * _L,), jnp.int32),
        pltpu.SemaphoreType.DMA,
        pltpu.SemaphoreType.DMA,
    ],
)
def _argmax_sc(x_hbm, out_hbm, buf0, buf1, res, gbuf, outv, shared,
               sem0, sem1):
    cid = lax.axis_index("c")
    sid = lax.axis_index("s")
    wid = cid * 16 + sid
    row0 = wid * _RPW
    bufs = (buf0, buf1)
    sems = (sem0, sem1)
    lanes = lax.iota(jnp.int32, _L)

    copies = [None, None]
    copies[0] = pltpu.async_copy(x_hbm.at[row0], buf0, sem0)
    resvec = jnp.zeros((_L,), jnp.int32)
    for r in range(_RPW):
        cur = r % 2
        nxt = (r + 1) % 2
        if r + 1 < _RPW:
            copies[nxt] = pltpu.async_copy(
                x_hbm.at[row0 + r + 1], bufs[nxt], sems[nxt])
        copies[cur].wait()
        a = _row_argmax(bufs[cur])
        resvec = jnp.where(lanes == (sid % _G) * _RPW + r, a, resvec)
    res[...] = resvec
    pltpu.sync_copy(res, shared.at[pl.ds(pl.multiple_of(sid * _L, _L), _L)])
    plsc.subcore_barrier()

    grp = (sid // _G) * _G
    pltpu.sync_copy(
        shared.at[pl.ds(pl.multiple_of(grp * _L, _G * _L), _G * _L)], gbuf)
    hi = lanes >> _GSHIFT
    combined = gbuf[pl.ds(0, _L)]
    for j in range(1, _G):
        rowj = gbuf[pl.ds(j * _L, _L)]
        combined = jnp.where(hi == j, rowj, combined)
    outv[...] = combined

    @pl.when(sid == grp)
    def _():
        off = pl.multiple_of((cid * 16 + sid) * _RPW, _L)
        pltpu.sync_copy(outv, out_hbm.at[pl.ds(off, _L)])


_TCR = 8
_TCB = 128
_TCACC = 4


def _tc_body(x_ref, o_ref):
    niter = x_ref.shape[1] // (_TCB * _TCACC)
    lane = lax.broadcasted_iota(jnp.int32, (_TCR, _TCB), 1)

    ms = [x_ref[:, pl.ds(k * _TCB, _TCB)] for k in range(_TCACC)]
    cs = [jnp.zeros((_TCR, _TCB), jnp.int32) for _ in range(_TCACC)]
    for t in range(1, niter):
        base = t * (_TCB * _TCACC)
        for k in range(_TCACC):
            x = x_ref[:, pl.ds(base + k * _TCB, _TCB)]
            gt = x > ms[k]
            ms[k] = jnp.where(gt, x, ms[k])
            cs[k] = jnp.where(gt, t, cs[k])

    stride = _TCACC * _TCB
    bm = ms[0]
    bidx = cs[0] * stride + lane
    for k in range(1, _TCACC):
        idxk = cs[k] * stride + (k * _TCB) + lane
        better = (ms[k] > bm) | ((ms[k] == bm) & (idxk < bidx))
        bm = jnp.where(better, ms[k], bm)
        bidx = jnp.where(better, idxk, bidx)
    rm = jnp.max(bm, axis=1)
    cand = jnp.where(bm == rm[:, None], bidx, jnp.int32(2 ** 30))
    o_ref[0, 0, :] = jnp.min(cand, axis=1)


def _argmax_tc(x, row_start, rows):
    grid = rows // _TCR
    blk0 = row_start // _TCR
    out = pl.pallas_call(
        _tc_body,
        grid=(grid,),
        in_specs=[pl.BlockSpec((_TCR, _C), lambda i: (i + blk0, 0))],
        out_specs=pl.BlockSpec((1, 1, _TCR), lambda i: (i, 0, 0)),
        out_shape=jax.ShapeDtypeStruct((grid, 1, _TCR), jnp.int32),
    )(x)
    return out.reshape(rows)


def kernel(tensor):
    sc_out = _argmax_sc(tensor)
    tc_out = _argmax_tc(tensor, _K_SC, _R - _K_SC)
    return jnp.concatenate(
        [sc_out.reshape(1, _K_SC), tc_out.reshape(1, _R - _K_SC)],
        axis=1).reshape(_R)


# speedup vs baseline: 1.0985x; 1.0889x over previous
import functools

import jax
import jax.numpy as jnp
from jax import lax
from jax.experimental import pallas as pl
from jax.experimental.pallas import tpu as pltpu
from jax.experimental.pallas import tpu_sc as plsc

_R = 128
_C = 32768
_L = 16
_NW = 32

_RPW = 2
_K_SC = _NW * _RPW
_G = 16 // _RPW
_GSHIFT = {1: 0, 2: 1, 4: 2}[_RPW]

_NCHUNK = _C // _L
_NACC = 4
_NITER = _NCHUNK // _NACC


def _row_argmax(buf):
    lanes = lax.iota(jnp.int32, _L)

    def step(t, carry):
        ms, cs = carry
        tvec = lax.broadcast(t, (_L,))
        base = t * (_L * _NACC)
        nms, ncs = [], []
        for k in range(_NACC):
            x = buf[pl.ds(base + k * _L, _L)]
            gt = x > ms[k]
            nms.append(jnp.where(gt, x, ms[k]))
            ncs.append(jnp.where(gt, tvec, cs[k]))
        return tuple(nms), tuple(ncs)

    ms0 = tuple(buf[pl.ds(k * _L, _L)] for k in range(_NACC))
    cs0 = tuple(jnp.zeros((_L,), jnp.int32) for _ in range(_NACC))
    ms, cs = lax.fori_loop(1, _NITER, step, (ms0, cs0), unroll=4)

    bm = ms[0]
    bidx = cs[0] * (_NACC * _L) + lanes
    for k in range(1, _NACC):
        idxk = cs[k] * (_NACC * _L) + (k * _L) + lanes
        better = (ms[k] > bm) | ((ms[k] == bm) & (idxk < bidx))
        bm = jnp.where(better, ms[k], bm)
        bidx = jnp.where(better, idxk, bidx)
    sm = bm[0]
    si = bidx[0]
    for j in range(1, _L):
        mv = bm[j]
        iv = bidx[j]
        better = (mv > sm) | ((mv == sm) & (iv < si))
        sm = jnp.where(better, mv, sm)
        si = jnp.where(better, iv, si)
    return si


@functools.partial(
    pl.kernel,
    mesh=plsc.VectorSubcoreMesh(core_axis_name="c", subcore_axis_name="s"),
    out_type=jax.ShapeDtypeStruct((_K_SC,), jnp.int32),
    scratch_types=[
        pltpu.VMEM((_C,), jnp.float32),
        pltpu.VMEM((_C,), jnp.float32),
        pltpu.VMEM((_L,), jnp.int32),
        pltpu.VMEM((_G * _L,), jnp.int32),
        pltpu.VMEM((_L,), jnp.int32),
        pltpu.VMEM_SHARED((16 * _L,), jnp.int32),
        pltpu.SemaphoreType.DMA,
        pltpu.SemaphoreType.DMA,
    ],
)
def _argmax_sc(x_hbm, out_hbm, buf0, buf1, res, gbuf, outv, shared,
               sem0, sem1):
    cid = lax.axis_index("c")
    sid = lax.axis_index("s")
    wid = cid * 16 + sid
    row0 = wid * _RPW
    bufs = (buf0, buf1)
    sems = (sem0, sem1)
    lanes = lax.iota(jnp.int32, _L)

    copies = [None, None]
    copies[0] = pltpu.async_copy(x_hbm.at[row0], buf0, sem0)
    resvec = jnp.zeros((_L,), jnp.int32)
    for r in range(_RPW):
        cur = r % 2
        nxt = (r + 1) % 2
        if r + 1 < _RPW:
            copies[nxt] = pltpu.async_copy(
                x_hbm.at[row0 + r + 1], bufs[nxt], sems[nxt])
        copies[cur].wait()
        a = _row_argmax(bufs[cur])
        resvec = jnp.where(lanes == (sid % _G) * _RPW + r, a, resvec)
    res[...] = resvec
    pltpu.sync_copy(res, shared.at[pl.ds(pl.multiple_of(sid * _L, _L), _L)])
    plsc.subcore_barrier()

    grp = (sid // _G) * _G
    pltpu.sync_copy(
        shared.at[pl.ds(pl.multiple_of(grp * _L, _G * _L), _G * _L)], gbuf)
    hi = lanes >> _GSHIFT
    combined = gbuf[pl.ds(0, _L)]
    for j in range(1, _G):
        rowj = gbuf[pl.ds(j * _L, _L)]
        combined = jnp.where(hi == j, rowj, combined)
    outv[...] = combined

    @pl.when(sid == grp)
    def _():
        off = pl.multiple_of((cid * 16 + sid) * _RPW, _L)
        pltpu.sync_copy(outv, out_hbm.at[pl.ds(off, _L)])


_TCR = 8
_TCB = 128
_TCACC = 4


def _tc_body(x_ref, o_ref):
    niter = x_ref.shape[1] // (_TCB * _TCACC)
    lane = lax.broadcasted_iota(jnp.int32, (_TCR, _TCB), 1)

    ms = [x_ref[:, pl.ds(k * _TCB, _TCB)] for k in range(_TCACC)]
    cs = [jnp.zeros((_TCR, _TCB), jnp.int32) for _ in range(_TCACC)]
    for t in range(1, niter):
        base = t * (_TCB * _TCACC)
        for k in range(_TCACC):
            x = x_ref[:, pl.ds(base + k * _TCB, _TCB)]
            gt = x > ms[k]
            ms[k] = jnp.where(gt, x, ms[k])
            cs[k] = jnp.where(gt, t, cs[k])

    stride = _TCACC * _TCB
    bm = ms[0]
    bidx = cs[0] * stride + lane
    for k in range(1, _TCACC):
        idxk = cs[k] * stride + (k * _TCB) + lane
        better = (ms[k] > bm) | ((ms[k] == bm) & (idxk < bidx))
        bm = jnp.where(better, ms[k], bm)
        bidx = jnp.where(better, idxk, bidx)
    rm = jnp.max(bm, axis=1)
    cand = jnp.where(bm == rm[:, None], bidx, jnp.int32(2 ** 30))
    o_ref[0, 0, :] = jnp.min(cand, axis=1)


def _argmax_tc(x, row_start, rows):
    grid = rows // _TCR
    blk0 = row_start // _TCR
    out = pl.pallas_call(
        _tc_body,
        grid=(grid,),
        in_specs=[pl.BlockSpec((_TCR, _C), lambda i: (i + blk0, 0))],
        out_specs=pl.BlockSpec((1, 1, _TCR), lambda i: (i, 0, 0)),
        out_shape=jax.ShapeDtypeStruct((grid, 1, _TCR), jnp.int32),
    )(x)
    return out.reshape(rows)


def kernel(tensor):
    sc_out = _argmax_sc(tensor)
    tc_out = _argmax_tc(tensor, _K_SC, _R - _K_SC)
    return jnp.concatenate(
        [sc_out.reshape(1, _K_SC), tc_out.reshape(1, _R - _K_SC)],
        axis=1).reshape(_R)
